# baseline (speedup 1.0000x reference)
import functools

import jax
import jax.numpy as jnp
from jax import lax
from jax.experimental import pallas as pl
from jax.experimental.pallas import tpu as pltpu
from jax.experimental.pallas import tpu_sc as plsc

VOCAB = 100000
EMBED_DIM = 64
BATCH = 1024
CTX = 50
CTX_PAD = 56

NUM_CORES = 2
NUM_SUBCORES = 16
NUM_WORKERS = NUM_CORES * NUM_SUBCORES
BPW = BATCH // NUM_WORKERS
LANES = 16
DVECS = EMBED_DIM // LANES

_sc_mesh = plsc.VectorSubcoreMesh(
    core_axis_name="c", subcore_axis_name="s",
    num_cores=NUM_CORES, num_subcores=NUM_SUBCORES)


@functools.partial(
    pl.kernel,
    out_type=jax.ShapeDtypeStruct((BATCH, EMBED_DIM), jnp.float32),
    mesh=_sc_mesh,
    scratch_types=[
        pltpu.VMEM((BPW, CTX_PAD), jnp.int32),
        pltpu.VMEM((CTX_PAD, EMBED_DIM), jnp.float32),
        pltpu.VMEM((CTX_PAD, EMBED_DIM), jnp.float32),
        pltpu.VMEM((BPW, EMBED_DIM), jnp.float32),
        pltpu.SemaphoreType.DMA,
        pltpu.SemaphoreType.DMA,
    ],
    compiler_params=pltpu.CompilerParams(use_tc_tiling_on_sc=False),
)
def _pool_sc(ctx_hbm, table_hbm, out_hbm, idx_v, rows0, rows1, pooled_v,
             sem0, sem1):
    wid = lax.axis_index("s") * NUM_CORES + lax.axis_index("c")
    base = wid * BPW
    pltpu.sync_copy(ctx_hbm.at[pl.ds(base, BPW)], idx_v)

    rows = (rows0, rows1)
    sems = (sem0, sem1)

    def gather(b, i):
        pltpu.make_async_copy(table_hbm.at[idx_v.at[b]], rows[i], sems[i]).start()

    def drain_and_pool(b, i):
        pltpu.make_async_copy(table_hbm.at[idx_v.at[b]], rows[i], sems[i]).wait()
        inv = jnp.float32(1.0 / CTX)
        for d in range(DVECS):
            acc = rows[i][0, pl.ds(d * LANES, LANES)]
            for c in range(1, CTX):
                acc = acc + rows[i][c, pl.ds(d * LANES, LANES)]
            pooled_v[b, pl.ds(d * LANES, LANES)] = acc * inv

    gather(0, 0)
    gather(1, 1)

    def body(t, carry):
        for i in range(2):
            b = 2 * t + i
            drain_and_pool(b, i)

            @pl.when(b + 2 < BPW)
            def _():
                gather(b + 2, i)
        return carry

    lax.fori_loop(0, BPW // 2, body, 0)
    pltpu.sync_copy(pooled_v, out_hbm.at[pl.ds(base, BPW)])


VTILE = 2048


def _proj_body(p_ref, w_ref, b_ref, o_ref):
    o_ref[...] = lax.dot_general(
        p_ref[...], w_ref[...],
        dimension_numbers=(((1,), (1,)), ((), ())),
        preferred_element_type=jnp.float32,
    ) + b_ref[...]


def _project(pooled, lin_w, lin_b2d):
    grid = (pl.cdiv(VOCAB, VTILE),)
    return pl.pallas_call(
        _proj_body,
        grid=grid,
        in_specs=[
            pl.BlockSpec((BATCH, EMBED_DIM), lambda j: (0, 0)),
            pl.BlockSpec((VTILE, EMBED_DIM), lambda j: (j, 0)),
            pl.BlockSpec((1, VTILE), lambda j: (0, j)),
        ],
        out_specs=pl.BlockSpec((BATCH, VTILE), lambda j: (0, j)),
        out_shape=jax.ShapeDtypeStruct((BATCH, VOCAB), jnp.float32),
    )(pooled, lin_w, lin_b2d)


def kernel(context, emb_table, lin_w, lin_b):
    ctx = context.astype(jnp.int32)
    ctx_pad = jnp.pad(ctx, ((0, 0), (0, CTX_PAD - CTX)))
    pooled = _pool_sc(ctx_pad, emb_table)
    return _project(pooled, lin_w, lin_b.reshape(1, VOCAB))
